# Initial kernel scaffold; baseline (speedup 1.0000x reference)
import jax
import jax.numpy as jnp
from jax.experimental import pallas as pl


def kernel(x, edge_index, W1, W2, comb_weight):
    raise NotImplementedError("write your pallas kernel here")


# speedup vs baseline: 5.5768x; 5.5768x over previous
import functools

import jax
import jax.numpy as jnp
from jax import lax
from jax.experimental import pallas as pl
from jax.experimental.pallas import tpu as pltpu
from jax.experimental.pallas import tpu_sc as plsc

N = 10000
E = 320000
F_IN = 128
HID = 64
NCLS = 64
K = 10

NT = 16
NC = 2
N_PAD = 10240
RPT = N_PAD // NT
F2 = HID // 2
EW = 80
PAD_ROW = N

_MESH = plsc.VectorSubcoreMesh(core_axis_name="c", subcore_axis_name="s")


def _zero16():
    return jnp.zeros((16,), jnp.float32)


def _bc(scalar, dtype=jnp.float32):
    return jnp.full((16,), scalar, dtype)


def _rsqrt16(dg):
    m = dg > 0.0
    xx = jnp.where(m, dg, 1.0)
    i = jax.lax.bitcast_convert_type(xx, jnp.int32)
    i = jnp.int32(0x5F3759DF) - jax.lax.shift_right_arithmetic(i, 1)
    y = jax.lax.bitcast_convert_type(i, jnp.float32)
    for _ in range(4):
        y = y * (1.5 - 0.5 * xx * y * y)
    return jnp.where(m, y, 0.0)


def _sc1_body(row1, col1, degp, colp, rowp, degs, rb, cb, wb, cpb, rpb, zb, ob):
    c = lax.axis_index("c")
    t = lax.axis_index("s")
    rbase = t * RPT

    for i in range(RPT // 16):
        zb[pl.ds(i * 16, 16)] = _zero16()
    pltpu.sync_copy(zb, degs.at[pl.ds(rbase, RPT)])
    plsc.subcore_barrier()

    per_tile = E // (NC * NT)
    e00 = c * (E // NC) + t * per_tile
    nunits = per_tile // EW

    pad16 = _bc(PAD_ROW, jnp.int32)
    npad16 = _bc(N_PAD, jnp.int32)

    def unit(uu, _):
        e0 = e00 + uu * EW
        pltpu.sync_copy(row1.at[pl.ds(e0, EW)], rb)
        pltpu.sync_copy(col1.at[pl.ds(e0, EW)], cb)
        for i in range(EW // 16):
            sl = pl.ds(i * 16, 16)
            r16 = rb[sl]
            c16 = cb[sl]
            m = r16 != c16
            wb[sl] = jnp.where(m, 1.0, 0.0)
            cpb[sl] = jnp.where(m, c16, pad16)
            rpb[sl] = r16 + npad16
        pltpu.sync_copy(cpb, colp.at[pl.ds(e0, EW)])
        pltpu.sync_copy(rpb, rowp.at[pl.ds(e0, EW)])
        pltpu.sync_copy(wb, degs.at[rb], add=True)
        return ()

    lax.fori_loop(0, nunits, unit, (), unroll=False)
    plsc.subcore_barrier()

    pltpu.sync_copy(degs.at[pl.ds(rbase, RPT)], ob)
    pltpu.sync_copy(ob, degp.at[pl.ds(c * N_PAD + rbase, RPT)])


_sc1 = functools.partial(
    pl.kernel,
    out_type=(
        jax.ShapeDtypeStruct((NC * N_PAD,), jnp.float32),
        jax.ShapeDtypeStruct((E,), jnp.int32),
        jax.ShapeDtypeStruct((E,), jnp.int32),
    ),
    mesh=_MESH,
    compiler_params=pltpu.CompilerParams(use_tc_tiling_on_sc=False),
    scratch_types=(
        pltpu.VMEM_SHARED((N_PAD,), jnp.float32),
        pltpu.VMEM((EW,), jnp.int32),
        pltpu.VMEM((EW,), jnp.int32),
        pltpu.VMEM((EW,), jnp.float32),
        pltpu.VMEM((EW,), jnp.int32),
        pltpu.VMEM((EW,), jnp.int32),
        pltpu.VMEM((RPT,), jnp.float32),
        pltpu.VMEM((RPT,), jnp.float32),
    ),
)(_sc1_body)


def _sc2_body(h2, degp, colp, rowp, cw, comb, ss,
              tx3, cmb, sv, gb, cb, rb, dv, d1v, cwv, sem):
    c = lax.axis_index("c")
    t = lax.axis_index("s")
    co = c * N_PAD
    rbase = t * RPT

    def zrow(i, _):
        sv[i, pl.ds(0, 16)] = _zero16()
        sv[i, pl.ds(16, 16)] = _zero16()
        return ()

    pltpu.sync_copy(cw, cwv)

    pltpu.sync_copy(degp.at[pl.ds(rbase, RPT)], dv)
    pltpu.sync_copy(degp.at[pl.ds(N_PAD + rbase, RPT)], d1v)

    def dinv_row(i, _):
        sl = pl.ds(i * 16, 16)
        dg = dv[sl] + d1v[sl]
        dv[sl] = _rsqrt16(dg)
        return ()
    lax.fori_loop(0, RPT // 16, dinv_row, (), unroll=False)

    pltpu.sync_copy(h2.at[pl.ds(co + rbase, RPT)], tx3.at[0])
    cvec = cwv[...]
    c0 = _bc(cvec[0])

    def init_row(i16, _):
        dvec = dv[pl.ds(i16 * 16, 16)]
        for j in range(16):
            i = i16 * 16 + j
            d16 = _bc(dvec[j])
            for hh in range(2):
                sl = pl.ds(hh * 16, 16)
                h16 = tx3[0, i, sl]
                cmb[i, sl] = c0 * h16
                sv[i, sl] = d16 * h16
        return ()
    lax.fori_loop(0, RPT // 16, init_row, (), unroll=False)
    pltpu.sync_copy(sv, ss.at[pl.ds(rbase, RPT)])
    lax.fori_loop(0, RPT, zrow, (), unroll=False)
    pltpu.sync_copy(sv, ss.at[pl.ds(N_PAD + rbase, RPT)])
    plsc.subcore_barrier()

    e00 = t * (E // NT)
    nunits = (E // NT) // EW

    def gather_scatter_phase():
        def unit(uu, _):
            e0 = e00 + uu * EW
            pltpu.sync_copy(colp.at[pl.ds(e0, EW)], cb)
            pltpu.sync_copy(rowp.at[pl.ds(e0, EW)], rb)
            pltpu.async_copy(ss.at[cb], gb, sem).wait()
            pltpu.sync_copy(gb, ss.at[rb], add=True)
            return ()
        lax.fori_loop(0, nunits, unit, (), unroll=False)
        plsc.subcore_barrier()

    def recurrence_phase(k, first):
        pltpu.sync_copy(ss.at[pl.ds(N_PAD + rbase, RPT)], sv)
        ck = jnp.take(cvec, jnp.full((16,), k, jnp.int32))
        p = jax.lax.rem(k, 2)

        def rowk(i16, _):
            dvec = dv[pl.ds(i16 * 16, 16)]
            for j in range(16):
                i = i16 * 16 + j
                d16 = _bc(dvec[j])
                for hh in range(2):
                    sl = pl.ds(hh * 16, 16)
                    if first:
                        tn = -(d16 * sv[i, sl])
                    else:
                        tn = -2.0 * (d16 * sv[i, sl]) - tx3[p, i, sl]
                    tx3[p, i, sl] = tn
                    sv[i, sl] = d16 * tn
                    cmb[i, sl] = cmb[i, sl] + ck * tn
            return ()
        lax.fori_loop(0, RPT // 16, rowk, (), unroll=False)
        pltpu.sync_copy(sv, ss.at[pl.ds(rbase, RPT)])
        lax.fori_loop(0, RPT, zrow, (), unroll=False)
        pltpu.sync_copy(sv, ss.at[pl.ds(N_PAD + rbase, RPT)])
        plsc.subcore_barrier()

    gather_scatter_phase()
    recurrence_phase(jnp.int32(1), True)

    def round_k(k, _):
        gather_scatter_phase()
        recurrence_phase(k, False)
        return ()
    lax.fori_loop(2, K + 1, round_k, (), unroll=False)

    pltpu.sync_copy(cmb, comb.at[pl.ds(co + rbase, RPT)])


_sc2 = functools.partial(
    pl.kernel,
    out_type=jax.ShapeDtypeStruct((NC * N_PAD, F2), jnp.float32),
    mesh=_MESH,
    compiler_params=pltpu.CompilerParams(use_tc_tiling_on_sc=False),
    scratch_types=(
        pltpu.VMEM_SHARED((2 * N_PAD, F2), jnp.float32),
        pltpu.VMEM((2, RPT, F2), jnp.float32),
        pltpu.VMEM((RPT, F2), jnp.float32),
        pltpu.VMEM((RPT, F2), jnp.float32),
        pltpu.VMEM((EW, F2), jnp.float32),
        pltpu.VMEM((EW,), jnp.int32),
        pltpu.VMEM((EW,), jnp.int32),
        pltpu.VMEM((RPT,), jnp.float32),
        pltpu.VMEM((RPT,), jnp.float32),
        pltpu.VMEM((16,), jnp.float32),
        pltpu.SemaphoreType.DMA,
    ),
)(_sc2_body)


def _tca_body(x_ref, w1_ref, o_ref):
    h = jnp.maximum(jnp.dot(x_ref[...], w1_ref[...],
                            preferred_element_type=jnp.float32), 0.0)
    o_ref[0] = h[:, :F2]
    o_ref[1] = h[:, F2:]


def _tca(x_pad, W1):
    R = 1024
    return pl.pallas_call(
        _tca_body,
        grid=(N_PAD // R,),
        in_specs=[
            pl.BlockSpec((R, F_IN), lambda r: (r, 0)),
            pl.BlockSpec((F_IN, HID), lambda r: (0, 0)),
        ],
        out_specs=pl.BlockSpec((2, R, F2), lambda r: (0, r, 0)),
        out_shape=jax.ShapeDtypeStruct((2, N_PAD, F2), jnp.float32),
    )(x_pad, W1)


def _tcb_body(lo_ref, hi_ref, w2_ref, o_ref):
    cb = jnp.concatenate([lo_ref[...], hi_ref[...]], axis=1)
    o_ref[...] = jnp.dot(cb, w2_ref[...], preferred_element_type=jnp.float32)


def _tcb(comb_flat, W2):
    R = 1024
    nb = N_PAD // R
    return pl.pallas_call(
        _tcb_body,
        grid=(nb,),
        in_specs=[
            pl.BlockSpec((R, F2), lambda r: (r, 0)),
            pl.BlockSpec((R, F2), lambda r, _nb=nb: (_nb + r, 0)),
            pl.BlockSpec((HID, NCLS), lambda r: (0, 0)),
        ],
        out_specs=pl.BlockSpec((R, NCLS), lambda r: (r, 0)),
        out_shape=jax.ShapeDtypeStruct((N_PAD, NCLS), jnp.float32),
    )(comb_flat, comb_flat, W2)


@jax.jit
def kernel(x, edge_index, W1, W2, comb_weight):
    row1 = edge_index[0]
    col1 = edge_index[1]
    x_pad = jnp.pad(x, ((0, N_PAD - N), (0, 0)))
    cw16 = jnp.pad(comb_weight, (0, 16 - (K + 1)))

    degp, colp, rowp = _sc1(row1, col1)
    h2 = _tca(x_pad, W1).reshape(NC * N_PAD, F2)
    comb = _sc2(h2, degp, colp, rowp, cw16)
    out = _tcb(comb, W2)
    return out[:N]
